# Initial kernel scaffold; baseline (speedup 1.0000x reference)
import jax
import jax.numpy as jnp
from jax.experimental import pallas as pl


def kernel(pred, gt):
    raise NotImplementedError("write your pallas kernel here")


# speedup vs baseline: 43.9455x; 43.9455x over previous
import functools

import jax
import jax.numpy as jnp
from jax.experimental import pallas as pl
from jax.experimental.pallas import tpu as pltpu

B = 4
N = 4096
K = 16
RT = 256
NSWEEP = 8
_INTMAX = 0x7FFFFFFF


def _knn_moments_kernel(pcol_ref, prow_ref, mom_ref):
    pc = pcol_ref[0]
    pr = prow_ref[0]
    g = jax.lax.dot_general(
        pr, pc, (((1,), (1,)), ((), ())),
        preferred_element_type=jnp.float32,
        precision=jax.lax.Precision.HIGHEST)
    pn_c = jnp.sum(pc * pc, axis=1)[None, :]
    pn_r = jnp.sum(pr * pr, axis=1)[:, None]
    d2 = jnp.maximum(pn_r + pn_c - 2.0 * g, 0.0)
    ki = jax.lax.bitcast_convert_type(d2, jnp.int32)
    col = jax.lax.broadcasted_iota(jnp.int32, (RT, N), 1)
    ki = (ki & jnp.int32(~0xFFF)) | col
    for _ in range(K):
        m = jnp.min(ki, axis=1, keepdims=True)
        ki = jnp.where(ki == m, _INTMAX, ki)
    mask = (ki == _INTMAX).astype(jnp.float32)
    x = pc[:, 0:1]
    y = pc[:, 1:2]
    z = pc[:, 2:3]
    one = jnp.ones_like(x)
    zero = jnp.zeros_like(x)
    mt = jnp.concatenate(
        [x, y, z, x * x, y * y, z * z, x * y, x * z, y * z,
         one, zero, zero, zero, zero, zero, zero], axis=1)
    mom_ref[0] = jax.lax.dot_general(
        mask, mt, (((1,), (0,)), ((), ())),
        preferred_element_type=jnp.float32,
        precision=jax.lax.Precision.HIGHEST)


def _moments(points8):
    grid = (points8.shape[0], N // RT)
    return pl.pallas_call(
        _knn_moments_kernel,
        grid=grid,
        in_specs=[
            pl.BlockSpec((1, N, 8), lambda b, t: (b, 0, 0)),
            pl.BlockSpec((1, RT, 8), lambda b, t: (b, t, 0)),
        ],
        out_specs=pl.BlockSpec((1, RT, 16), lambda b, t: (b, t, 0)),
        out_shape=jax.ShapeDtypeStruct((points8.shape[0], N, 16),
                                       jnp.float32),
    )(points8, points8)


def _rotate(A, V, p, q):
    app, aqq, apq = A[(p, p)], A[(q, q)], A[(p, q)]
    safe = jnp.where(apq == 0.0, 1.0, apq)
    tau = (aqq - app) / (2.0 * safe)
    sgn = jnp.where(tau >= 0.0, 1.0, -1.0)
    t = sgn / (jnp.abs(tau) + jnp.sqrt(1.0 + tau * tau))
    t = jnp.where(apq == 0.0, 0.0, t)
    c = jax.lax.rsqrt(1.0 + t * t)
    s = t * c
    r = ({0, 1, 2} - {p, q}).pop()

    def key(i, j):
        return (i, j) if i <= j else (j, i)

    apr, aqr = A[key(p, r)], A[key(q, r)]
    A[(p, p)] = app - t * apq
    A[(q, q)] = aqq + t * apq
    A[(p, q)] = jnp.zeros_like(apq)
    A[key(p, r)] = c * apr - s * aqr
    A[key(q, r)] = s * apr + c * aqr
    for i in range(3):
        vip, viq = V[(i, p)], V[(i, q)]
        V[(i, p)] = c * vip - s * viq
        V[(i, q)] = s * vip + c * viq


def _normals_from_moments(m_ref):
    inv_k = 1.0 / K
    sx, sy, sz = m_ref[0], m_ref[1], m_ref[2]
    sxx, syy, szz = m_ref[3], m_ref[4], m_ref[5]
    sxy, sxz, syz = m_ref[6], m_ref[7], m_ref[8]
    mx, my, mz = sx * inv_k, sy * inv_k, sz * inv_k
    A = {
        (0, 0): sxx * inv_k - mx * mx,
        (1, 1): syy * inv_k - my * my,
        (2, 2): szz * inv_k - mz * mz,
        (0, 1): sxy * inv_k - mx * my,
        (0, 2): sxz * inv_k - mx * mz,
        (1, 2): syz * inv_k - my * mz,
    }
    one = jnp.ones_like(sx)
    zero = jnp.zeros_like(sx)
    V = {(i, j): (one if i == j else zero)
         for i in range(3) for j in range(3)}
    for _ in range(NSWEEP):
        for (p, q) in ((0, 2), (1, 2), (0, 1)):
            _rotate(A, V, p, q)
    d0, d1, d2 = A[(0, 0)], A[(1, 1)], A[(2, 2)]
    take0 = (d0 <= d1) & (d0 <= d2)
    take1 = jnp.logical_not(take0) & (d1 <= d2)

    def pick(i):
        return jnp.where(take0, V[(i, 0)],
                         jnp.where(take1, V[(i, 1)], V[(i, 2)]))

    nx, ny, nz = pick(0), pick(1), pick(2)
    nrm = jnp.sqrt(nx * nx + ny * ny + nz * nz) + 1e-12
    return nx / nrm, ny / nrm, nz / nrm


def _loss_kernel(mp_ref, mg_ref, out_ref):
    px, py, pz = _normals_from_moments(mp_ref)
    gx, gy, gz = _normals_from_moments(mg_ref)
    dot = px * gx + py * gy + pz * gz
    out_ref[0, 0] = 1.0 - jnp.sum(dot) / (B * N)


def _loss(mom_p, mom_g):
    out = pl.pallas_call(
        _loss_kernel,
        out_specs=pl.BlockSpec(memory_space=pltpu.SMEM),
        out_shape=jax.ShapeDtypeStruct((1, 1), jnp.float32),
    )(mom_p, mom_g)
    return out.reshape(())


@jax.jit
def kernel(pred, gt):
    pts = jnp.concatenate([pred, gt], axis=0)
    pts8 = jnp.pad(pts, ((0, 0), (0, 0), (0, 5)))
    mom = _moments(pts8)
    mom2 = mom.reshape(2, B * N, 16).transpose(0, 2, 1)
    mom2 = mom2.reshape(2, 16, 128, 128)
    return _loss(mom2[0], mom2[1])


# speedup vs baseline: 44.4642x; 1.0118x over previous
import functools

import jax
import jax.numpy as jnp
from jax.experimental import pallas as pl
from jax.experimental.pallas import tpu as pltpu

B = 4
N = 4096
K = 16
RT = 256
NSWEEP = 8
_INTMAX = 0x7FFFFFFF


def _knn_moments_kernel(pcol_ref, prow_ref, mom_ref):
    pc = pcol_ref[0]
    pr = prow_ref[0]
    g = jax.lax.dot_general(
        pr, pc, (((1,), (1,)), ((), ())),
        preferred_element_type=jnp.float32,
        precision=jax.lax.Precision.HIGHEST)
    pn_c = jnp.sum(pc * pc, axis=1)[None, :]
    pn_r = jnp.sum(pr * pr, axis=1)[:, None]
    d2 = jnp.maximum(pn_r + pn_c - 2.0 * g, 0.0)
    ki = jax.lax.bitcast_convert_type(d2, jnp.int32)
    col = jax.lax.broadcasted_iota(jnp.int32, (RT, N), 1)
    ki = (ki & jnp.int32(~0xFFF)) | col
    thr = jnp.full((RT, 1), -1, jnp.int32)
    for _ in range(K):
        elig = jnp.where(ki > thr, ki, _INTMAX)
        thr = jnp.min(elig, axis=1, keepdims=True)
    mask = (ki <= thr).astype(jnp.float32)
    x = pc[:, 0:1]
    y = pc[:, 1:2]
    z = pc[:, 2:3]
    one = jnp.ones_like(x)
    zero = jnp.zeros_like(x)
    mt = jnp.concatenate(
        [x, y, z, x * x, y * y, z * z, x * y, x * z, y * z,
         one, zero, zero, zero, zero, zero, zero], axis=1)
    mom_ref[0] = jax.lax.dot_general(
        mask, mt, (((1,), (0,)), ((), ())),
        preferred_element_type=jnp.float32,
        precision=jax.lax.Precision.HIGHEST)


def _moments(points8):
    grid = (points8.shape[0], N // RT)
    return pl.pallas_call(
        _knn_moments_kernel,
        grid=grid,
        in_specs=[
            pl.BlockSpec((1, N, 8), lambda b, t: (b, 0, 0)),
            pl.BlockSpec((1, RT, 8), lambda b, t: (b, t, 0)),
        ],
        out_specs=pl.BlockSpec((1, RT, 16), lambda b, t: (b, t, 0)),
        out_shape=jax.ShapeDtypeStruct((points8.shape[0], N, 16),
                                       jnp.float32),
    )(points8, points8)


def _rotate(A, V, p, q):
    app, aqq, apq = A[(p, p)], A[(q, q)], A[(p, q)]
    safe = jnp.where(apq == 0.0, 1.0, apq)
    tau = (aqq - app) / (2.0 * safe)
    sgn = jnp.where(tau >= 0.0, 1.0, -1.0)
    t = sgn / (jnp.abs(tau) + jnp.sqrt(1.0 + tau * tau))
    t = jnp.where(apq == 0.0, 0.0, t)
    c = jax.lax.rsqrt(1.0 + t * t)
    s = t * c
    r = ({0, 1, 2} - {p, q}).pop()

    def key(i, j):
        return (i, j) if i <= j else (j, i)

    apr, aqr = A[key(p, r)], A[key(q, r)]
    A[(p, p)] = app - t * apq
    A[(q, q)] = aqq + t * apq
    A[(p, q)] = jnp.zeros_like(apq)
    A[key(p, r)] = c * apr - s * aqr
    A[key(q, r)] = s * apr + c * aqr
    for i in range(3):
        vip, viq = V[(i, p)], V[(i, q)]
        V[(i, p)] = c * vip - s * viq
        V[(i, q)] = s * vip + c * viq


def _normals_from_moments(m_ref):
    inv_k = 1.0 / K
    sx, sy, sz = m_ref[0], m_ref[1], m_ref[2]
    sxx, syy, szz = m_ref[3], m_ref[4], m_ref[5]
    sxy, sxz, syz = m_ref[6], m_ref[7], m_ref[8]
    mx, my, mz = sx * inv_k, sy * inv_k, sz * inv_k
    A = {
        (0, 0): sxx * inv_k - mx * mx,
        (1, 1): syy * inv_k - my * my,
        (2, 2): szz * inv_k - mz * mz,
        (0, 1): sxy * inv_k - mx * my,
        (0, 2): sxz * inv_k - mx * mz,
        (1, 2): syz * inv_k - my * mz,
    }
    one = jnp.ones_like(sx)
    zero = jnp.zeros_like(sx)
    V = {(i, j): (one if i == j else zero)
         for i in range(3) for j in range(3)}
    for _ in range(NSWEEP):
        for (p, q) in ((0, 2), (1, 2), (0, 1)):
            _rotate(A, V, p, q)
    d0, d1, d2 = A[(0, 0)], A[(1, 1)], A[(2, 2)]
    take0 = (d0 <= d1) & (d0 <= d2)
    take1 = jnp.logical_not(take0) & (d1 <= d2)

    def pick(i):
        return jnp.where(take0, V[(i, 0)],
                         jnp.where(take1, V[(i, 1)], V[(i, 2)]))

    nx, ny, nz = pick(0), pick(1), pick(2)
    nrm = jnp.sqrt(nx * nx + ny * ny + nz * nz) + 1e-12
    return nx / nrm, ny / nrm, nz / nrm


def _loss_kernel(mp_ref, mg_ref, out_ref):
    px, py, pz = _normals_from_moments(mp_ref)
    gx, gy, gz = _normals_from_moments(mg_ref)
    dot = px * gx + py * gy + pz * gz
    out_ref[0, 0] = 1.0 - jnp.sum(dot) / (B * N)


def _loss(mom_p, mom_g):
    out = pl.pallas_call(
        _loss_kernel,
        out_specs=pl.BlockSpec(memory_space=pltpu.SMEM),
        out_shape=jax.ShapeDtypeStruct((1, 1), jnp.float32),
    )(mom_p, mom_g)
    return out.reshape(())


@jax.jit
def kernel(pred, gt):
    pts = jnp.concatenate([pred, gt], axis=0)
    pts8 = jnp.pad(pts, ((0, 0), (0, 0), (0, 5)))
    mom = _moments(pts8)
    mom2 = mom.reshape(2, B * N, 16).transpose(0, 2, 1)
    mom2 = mom2.reshape(2, 16, 128, 128)
    return _loss(mom2[0], mom2[1])


# speedup vs baseline: 51.2239x; 1.1520x over previous
import functools

import jax
import jax.numpy as jnp
from jax.experimental import pallas as pl
from jax.experimental.pallas import tpu as pltpu

B = 4
N = 4096
K = 16
RT = 256
NSWEEP = 8
_INTMAX = 0x7FFFFFFF


def _knn_moments_kernel(pcol_ref, prow_ref, mom_ref, mt_ref):
    pc = pcol_ref[0]
    pr = prow_ref[0]

    @pl.when(pl.program_id(1) == 0)
    def _build_moment_table():
        x = pc[:, 0:1]
        y = pc[:, 1:2]
        z = pc[:, 2:3]
        one = jnp.ones_like(x)
        zero = jnp.zeros_like(x)
        mt_ref[...] = jnp.concatenate(
            [x, y, z, x * x, y * y, z * z, x * y, x * z, y * z,
             one, zero, zero, zero, zero, zero, zero], axis=1)

    g = jax.lax.dot_general(
        pr, pc, (((1,), (1,)), ((), ())),
        preferred_element_type=jnp.float32,
        precision=jax.lax.Precision.HIGHEST)
    pn_c = jnp.sum(pc * pc, axis=1)[None, :]
    pn_r = jnp.sum(pr * pr, axis=1)[:, None]
    d2 = jnp.maximum(pn_r + pn_c - 2.0 * g, 0.0)
    ki = jax.lax.bitcast_convert_type(d2, jnp.int32)
    col = jax.lax.broadcasted_iota(jnp.int32, (RT, N), 1)
    kf = jax.lax.bitcast_convert_type((ki & jnp.int32(~0xFFF)) | col,
                                      jnp.float32)
    inf = jnp.float32(jnp.inf)
    thr = jnp.full((RT, 1), -1.0, jnp.float32)
    for _ in range(K):
        elig = jnp.where(kf > thr, kf, inf)
        thr = jnp.min(elig, axis=1, keepdims=True)
    mask = (kf <= thr).astype(jnp.float32)
    mom_ref[0] = jax.lax.dot_general(
        mask, mt_ref[...], (((1,), (0,)), ((), ())),
        preferred_element_type=jnp.float32,
        precision=jax.lax.Precision.HIGHEST)


def _moments(points8):
    grid = (points8.shape[0], N // RT)
    return pl.pallas_call(
        _knn_moments_kernel,
        grid=grid,
        in_specs=[
            pl.BlockSpec((1, N, 8), lambda b, t: (b, 0, 0)),
            pl.BlockSpec((1, RT, 8), lambda b, t: (b, t, 0)),
        ],
        out_specs=pl.BlockSpec((1, RT, 16), lambda b, t: (b, t, 0)),
        out_shape=jax.ShapeDtypeStruct((points8.shape[0], N, 16),
                                       jnp.float32),
        scratch_shapes=[pltpu.VMEM((N, 16), jnp.float32)],
    )(points8, points8)


def _rotate(A, V, p, q):
    app, aqq, apq = A[(p, p)], A[(q, q)], A[(p, q)]
    safe = jnp.where(apq == 0.0, 1.0, apq)
    tau = (aqq - app) / (2.0 * safe)
    sgn = jnp.where(tau >= 0.0, 1.0, -1.0)
    t = sgn / (jnp.abs(tau) + jnp.sqrt(1.0 + tau * tau))
    t = jnp.where(apq == 0.0, 0.0, t)
    c = jax.lax.rsqrt(1.0 + t * t)
    s = t * c
    r = ({0, 1, 2} - {p, q}).pop()

    def key(i, j):
        return (i, j) if i <= j else (j, i)

    apr, aqr = A[key(p, r)], A[key(q, r)]
    A[(p, p)] = app - t * apq
    A[(q, q)] = aqq + t * apq
    A[(p, q)] = jnp.zeros_like(apq)
    A[key(p, r)] = c * apr - s * aqr
    A[key(q, r)] = s * apr + c * aqr
    for i in range(3):
        vip, viq = V[(i, p)], V[(i, q)]
        V[(i, p)] = c * vip - s * viq
        V[(i, q)] = s * vip + c * viq


def _normals_from_moments(m_ref):
    inv_k = 1.0 / K
    sx, sy, sz = m_ref[0], m_ref[1], m_ref[2]
    sxx, syy, szz = m_ref[3], m_ref[4], m_ref[5]
    sxy, sxz, syz = m_ref[6], m_ref[7], m_ref[8]
    mx, my, mz = sx * inv_k, sy * inv_k, sz * inv_k
    A = {
        (0, 0): sxx * inv_k - mx * mx,
        (1, 1): syy * inv_k - my * my,
        (2, 2): szz * inv_k - mz * mz,
        (0, 1): sxy * inv_k - mx * my,
        (0, 2): sxz * inv_k - mx * mz,
        (1, 2): syz * inv_k - my * mz,
    }
    one = jnp.ones_like(sx)
    zero = jnp.zeros_like(sx)
    V = {(i, j): (one if i == j else zero)
         for i in range(3) for j in range(3)}
    for _ in range(NSWEEP):
        for (p, q) in ((0, 2), (1, 2), (0, 1)):
            _rotate(A, V, p, q)
    d0, d1, d2 = A[(0, 0)], A[(1, 1)], A[(2, 2)]
    take0 = (d0 <= d1) & (d0 <= d2)
    take1 = jnp.logical_not(take0) & (d1 <= d2)

    def pick(i):
        return jnp.where(take0, V[(i, 0)],
                         jnp.where(take1, V[(i, 1)], V[(i, 2)]))

    nx, ny, nz = pick(0), pick(1), pick(2)
    nrm = jnp.sqrt(nx * nx + ny * ny + nz * nz) + 1e-12
    return nx / nrm, ny / nrm, nz / nrm


def _loss_kernel(mp_ref, mg_ref, out_ref):
    px, py, pz = _normals_from_moments(mp_ref)
    gx, gy, gz = _normals_from_moments(mg_ref)
    dot = px * gx + py * gy + pz * gz
    out_ref[0, 0] = 1.0 - jnp.sum(dot) / (B * N)


def _loss(mom_p, mom_g):
    out = pl.pallas_call(
        _loss_kernel,
        out_specs=pl.BlockSpec(memory_space=pltpu.SMEM),
        out_shape=jax.ShapeDtypeStruct((1, 1), jnp.float32),
    )(mom_p, mom_g)
    return out.reshape(())


@jax.jit
def kernel(pred, gt):
    pts = jnp.concatenate([pred, gt], axis=0)
    pts8 = jnp.pad(pts, ((0, 0), (0, 0), (0, 5)))
    mom = _moments(pts8)
    mom2 = mom.reshape(2, B * N, 16).transpose(0, 2, 1)
    mom2 = mom2.reshape(2, 16, 128, 128)
    return _loss(mom2[0], mom2[1])


# speedup vs baseline: 64.7157x; 1.2634x over previous
import functools

import jax
import jax.numpy as jnp
from jax.experimental import pallas as pl
from jax.experimental.pallas import tpu as pltpu

B = 4
N = 4096
K = 16
RT = 256
NSWEEP = 8
_INTMAX = 0x7FFFFFFF


def _knn_moments_kernel(pcol_ref, prow_ref, mom_ref, mt_ref, thr_ref):
    pc = pcol_ref[0]
    pr = prow_ref[0]

    @pl.when(pl.program_id(1) == 0)
    def _build_moment_table():
        x = pc[:, 0:1]
        y = pc[:, 1:2]
        z = pc[:, 2:3]
        one = jnp.ones_like(x)
        zero = jnp.zeros_like(x)
        mt_ref[...] = jnp.concatenate(
            [x, y, z, x * x, y * y, z * z, x * y, x * z, y * z,
             one, zero, zero, zero, zero, zero, zero], axis=1)

    g = jax.lax.dot_general(
        pr, pc, (((1,), (1,)), ((), ())),
        preferred_element_type=jnp.float32,
        precision=jax.lax.Precision.HIGHEST)
    pn_c = jnp.sum(pc * pc, axis=1)[None, :]
    pn_r = jnp.sum(pr * pr, axis=1)[:, None]
    d2 = jnp.maximum(pn_r + pn_c - 2.0 * g, 0.0)
    ki = jax.lax.bitcast_convert_type(d2, jnp.int32)
    col = jax.lax.broadcasted_iota(jnp.int32, (RT, N), 1)
    kf = jax.lax.bitcast_convert_type((ki & jnp.int32(~0xFFF)) | col,
                                      jnp.float32)
    inf = jnp.float32(jnp.inf)
    m1 = jnp.full((RT, 128), inf, jnp.float32)
    m2, m3, m4 = m1, m1, m1
    for j in range(N // 128):
        v = kf[:, j * 128:(j + 1) * 128]
        x = jnp.maximum(m1, v)
        m1 = jnp.minimum(m1, v)
        x, m2 = jnp.maximum(m2, x), jnp.minimum(m2, x)
        x, m3 = jnp.maximum(m3, x), jnp.minimum(m3, x)
        m4 = jnp.minimum(m4, x)
    thr = jnp.full((RT, 1), -1.0, jnp.float32)
    for _ in range(K):
        cur = jnp.where(m4 > thr, m4, inf)
        cur = jnp.where(m3 > thr, m3, cur)
        cur = jnp.where(m2 > thr, m2, cur)
        cur = jnp.where(m1 > thr, m1, cur)
        thr = jnp.min(cur, axis=1, keepdims=True)
    cnt = jnp.sum((kf <= thr).astype(jnp.float32), axis=1, keepdims=True)
    thr_ref[...] = thr

    @pl.when(jnp.any(cnt != float(K)))
    def _exact_fallback():
        t = jnp.full((RT, 1), -1.0, jnp.float32)
        for _ in range(K):
            elig = jnp.where(kf > t, kf, inf)
            t = jnp.min(elig, axis=1, keepdims=True)
        thr_ref[...] = t

    mask = (kf <= thr_ref[...]).astype(jnp.float32)
    mom_ref[0] = jax.lax.dot_general(
        mask, mt_ref[...], (((1,), (0,)), ((), ())),
        preferred_element_type=jnp.float32,
        precision=jax.lax.Precision.HIGHEST)


def _moments(points8):
    grid = (points8.shape[0], N // RT)
    return pl.pallas_call(
        _knn_moments_kernel,
        grid=grid,
        in_specs=[
            pl.BlockSpec((1, N, 8), lambda b, t: (b, 0, 0)),
            pl.BlockSpec((1, RT, 8), lambda b, t: (b, t, 0)),
        ],
        out_specs=pl.BlockSpec((1, RT, 16), lambda b, t: (b, t, 0)),
        out_shape=jax.ShapeDtypeStruct((points8.shape[0], N, 16),
                                       jnp.float32),
        scratch_shapes=[pltpu.VMEM((N, 16), jnp.float32),
                        pltpu.VMEM((RT, 1), jnp.float32)],
    )(points8, points8)


def _rotate(A, V, p, q):
    app, aqq, apq = A[(p, p)], A[(q, q)], A[(p, q)]
    safe = jnp.where(apq == 0.0, 1.0, apq)
    tau = (aqq - app) / (2.0 * safe)
    sgn = jnp.where(tau >= 0.0, 1.0, -1.0)
    t = sgn / (jnp.abs(tau) + jnp.sqrt(1.0 + tau * tau))
    t = jnp.where(apq == 0.0, 0.0, t)
    c = jax.lax.rsqrt(1.0 + t * t)
    s = t * c
    r = ({0, 1, 2} - {p, q}).pop()

    def key(i, j):
        return (i, j) if i <= j else (j, i)

    apr, aqr = A[key(p, r)], A[key(q, r)]
    A[(p, p)] = app - t * apq
    A[(q, q)] = aqq + t * apq
    A[(p, q)] = jnp.zeros_like(apq)
    A[key(p, r)] = c * apr - s * aqr
    A[key(q, r)] = s * apr + c * aqr
    for i in range(3):
        vip, viq = V[(i, p)], V[(i, q)]
        V[(i, p)] = c * vip - s * viq
        V[(i, q)] = s * vip + c * viq


def _normals_from_moments(m_ref):
    inv_k = 1.0 / K
    sx, sy, sz = m_ref[0], m_ref[1], m_ref[2]
    sxx, syy, szz = m_ref[3], m_ref[4], m_ref[5]
    sxy, sxz, syz = m_ref[6], m_ref[7], m_ref[8]
    mx, my, mz = sx * inv_k, sy * inv_k, sz * inv_k
    A = {
        (0, 0): sxx * inv_k - mx * mx,
        (1, 1): syy * inv_k - my * my,
        (2, 2): szz * inv_k - mz * mz,
        (0, 1): sxy * inv_k - mx * my,
        (0, 2): sxz * inv_k - mx * mz,
        (1, 2): syz * inv_k - my * mz,
    }
    one = jnp.ones_like(sx)
    zero = jnp.zeros_like(sx)
    V = {(i, j): (one if i == j else zero)
         for i in range(3) for j in range(3)}
    for _ in range(NSWEEP):
        for (p, q) in ((0, 2), (1, 2), (0, 1)):
            _rotate(A, V, p, q)
    d0, d1, d2 = A[(0, 0)], A[(1, 1)], A[(2, 2)]
    take0 = (d0 <= d1) & (d0 <= d2)
    take1 = jnp.logical_not(take0) & (d1 <= d2)

    def pick(i):
        return jnp.where(take0, V[(i, 0)],
                         jnp.where(take1, V[(i, 1)], V[(i, 2)]))

    nx, ny, nz = pick(0), pick(1), pick(2)
    nrm = jnp.sqrt(nx * nx + ny * ny + nz * nz) + 1e-12
    return nx / nrm, ny / nrm, nz / nrm


def _loss_kernel(mp_ref, mg_ref, out_ref):
    px, py, pz = _normals_from_moments(mp_ref)
    gx, gy, gz = _normals_from_moments(mg_ref)
    dot = px * gx + py * gy + pz * gz
    out_ref[0, 0] = 1.0 - jnp.sum(dot) / (B * N)


def _loss(mom_p, mom_g):
    out = pl.pallas_call(
        _loss_kernel,
        out_specs=pl.BlockSpec(memory_space=pltpu.SMEM),
        out_shape=jax.ShapeDtypeStruct((1, 1), jnp.float32),
    )(mom_p, mom_g)
    return out.reshape(())


@jax.jit
def kernel(pred, gt):
    pts = jnp.concatenate([pred, gt], axis=0)
    pts8 = jnp.pad(pts, ((0, 0), (0, 0), (0, 5)))
    mom = _moments(pts8)
    mom2 = mom.reshape(2, B * N, 16).transpose(0, 2, 1)
    mom2 = mom2.reshape(2, 16, 128, 128)
    return _loss(mom2[0], mom2[1])


# speedup vs baseline: 130.8111x; 2.0213x over previous
import functools

import jax
import jax.numpy as jnp
from jax import lax
from jax.experimental import pallas as pl
from jax.experimental.pallas import tpu as pltpu
from jax.experimental.pallas import tpu_sc as plsc

B = 4
N = 4096
K = 16
RT = 256
NSWEEP = 8
NW = 32
M_ALL = 2 * B * N
ROWS_PER = M_ALL // NW


def _knn_idx_kernel(pcol_ref, prow_ref, idx_ref, mt_ref, pn_ref):
    pc = pcol_ref[0]
    pr = prow_ref[0]
    t_id = pl.program_id(1)

    @pl.when(t_id == 0)
    def _build_moment_table():
        x = pc[:, 0:1]
        y = pc[:, 1:2]
        z = pc[:, 2:3]
        one = jnp.ones_like(x)
        zero = jnp.zeros_like(x)
        mt_ref[0] = jnp.concatenate(
            [x, y, z, x * x, y * y, z * z, x * y, x * z, y * z,
             one, zero, zero, zero, zero, zero, zero], axis=1)
        pn_ref[...] = jnp.sum(pc * pc, axis=1)[None, :]

    g = jax.lax.dot_general(
        pr, pc, (((1,), (1,)), ((), ())),
        preferred_element_type=jnp.float32,
        precision=jax.lax.Precision.HIGHEST)
    pn_c = pn_ref[...]
    pn_r = jnp.sum(pr * pr, axis=1)[:, None]
    d2 = jnp.maximum(pn_r + pn_c - 2.0 * g, 2e-38)
    ki = jax.lax.bitcast_convert_type(d2, jnp.int32)
    col = jax.lax.broadcasted_iota(jnp.int32, (RT, N), 1)
    kf = jax.lax.bitcast_convert_type((ki & jnp.int32(~0xFFF)) | col,
                                      jnp.float32)
    gbase = pl.program_id(0) * N

    def decode_store(t, thr):
        ii = jax.lax.bitcast_convert_type(thr, jnp.int32) & 0xFFF
        idx_ref[0, :, t:t + 1] = ii + gbase

    inf = jnp.float32(jnp.inf)
    m1 = jnp.full((RT, 128), inf, jnp.float32)
    m2, m3, m4 = m1, m1, m1
    for j in range(N // 128):
        v = kf[:, j * 128:(j + 1) * 128]
        x = jnp.maximum(m1, v)
        m1 = jnp.minimum(m1, v)
        x, m2 = jnp.maximum(m2, x), jnp.minimum(m2, x)
        x, m3 = jnp.maximum(m3, x), jnp.minimum(m3, x)
        m4 = jnp.minimum(m4, x)
    thr = jnp.full((RT, 1), -1.0, jnp.float32)
    for t in range(K):
        cur = jnp.where(m4 > thr, m4, inf)
        cur = jnp.where(m3 > thr, m3, cur)
        cur = jnp.where(m2 > thr, m2, cur)
        cur = jnp.where(m1 > thr, m1, cur)
        thr = jnp.min(cur, axis=1, keepdims=True)
        decode_store(t, thr)
    cnt = jnp.sum((kf <= thr).astype(jnp.float32), axis=1, keepdims=True)

    @pl.when(jnp.any(cnt != float(K)))
    def _exact_fallback():
        t2 = jnp.full((RT, 1), -1.0, jnp.float32)
        for t in range(K):
            elig = jnp.where(kf > t2, kf, inf)
            t2 = jnp.min(elig, axis=1, keepdims=True)
            decode_store(t, t2)


def _knn_idx(points8):
    grid = (points8.shape[0], N // RT)
    return pl.pallas_call(
        _knn_idx_kernel,
        grid=grid,
        in_specs=[
            pl.BlockSpec((1, N, 8), lambda b, t: (b, 0, 0)),
            pl.BlockSpec((1, RT, 8), lambda b, t: (b, t, 0)),
        ],
        out_specs=[
            pl.BlockSpec((1, RT, 16), lambda b, t: (b, t, 0)),
            pl.BlockSpec((1, N, 16), lambda b, t: (b, 0, 0)),
        ],
        out_shape=[
            jax.ShapeDtypeStruct((points8.shape[0], N, 16), jnp.int32),
            jax.ShapeDtypeStruct((points8.shape[0], N, 16), jnp.float32),
        ],
        scratch_shapes=[pltpu.VMEM((1, N), jnp.float32)],
    )(points8, points8)


def _sc_moments(mtab, idx3):
    mesh = plsc.VectorSubcoreMesh(core_axis_name="c", subcore_axis_name="s")

    n_chunks = ROWS_PER * 16 // 128

    @functools.partial(
        pl.kernel,
        out_type=jax.ShapeDtypeStruct((M_ALL, 16), jnp.float32),
        mesh=mesh,
        scratch_types=[
            pltpu.VMEM((128, 128), jnp.int32),
            pltpu.VMEM((128, 16), jnp.float32),
            pltpu.VMEM((128, 16), jnp.float32),
            pltpu.VMEM((ROWS_PER, 16), jnp.float32),
            pltpu.SemaphoreType.DMA,
            pltpu.SemaphoreType.DMA,
        ],
        compiler_params=pltpu.CompilerParams(use_tc_tiling_on_sc=False),
    )
    def k(mtab_hbm, idx_hbm, out_hbm, idxv, buf0, buf1, obuf, sem0, sem1):
        wid = lax.axis_index("s") * 2 + lax.axis_index("c")
        base = wid * ROWS_PER
        pltpu.sync_copy(idx_hbm.at[wid], idxv)
        bufs = (buf0, buf1)
        sems = (sem0, sem1)

        def gather(g, b):
            pltpu.make_async_copy(
                mtab_hbm.at[idxv.at[g]], bufs[b], sems[b]).start()

        def accumulate(g, b):
            buf = bufs[b]
            pltpu.make_async_copy(
                mtab_hbm.at[idxv.at[g]], buf, sems[b]).wait()
            for p in range(8):
                acc = buf[p * 16]
                for t in range(1, 16):
                    acc = acc + buf[p * 16 + t]
                obuf[g * 8 + p] = acc

        def body(i, _):
            g = 2 * i
            gather(g, 0)
            gather(g + 1, 1)
            accumulate(g, 0)
            accumulate(g + 1, 1)
            return 0

        lax.fori_loop(0, n_chunks // 2, body, 0)
        pltpu.sync_copy(obuf, out_hbm.at[pl.ds(base, ROWS_PER)])

    return k(mtab, idx3)


def _rotate(A, V, p, q):
    app, aqq, apq = A[(p, p)], A[(q, q)], A[(p, q)]
    safe = jnp.where(apq == 0.0, 1.0, apq)
    tau = (aqq - app) / (2.0 * safe)
    sgn = jnp.where(tau >= 0.0, 1.0, -1.0)
    t = sgn / (jnp.abs(tau) + jnp.sqrt(1.0 + tau * tau))
    t = jnp.where(apq == 0.0, 0.0, t)
    c = jax.lax.rsqrt(1.0 + t * t)
    s = t * c
    r = ({0, 1, 2} - {p, q}).pop()

    def key(i, j):
        return (i, j) if i <= j else (j, i)

    apr, aqr = A[key(p, r)], A[key(q, r)]
    A[(p, p)] = app - t * apq
    A[(q, q)] = aqq + t * apq
    A[(p, q)] = jnp.zeros_like(apq)
    A[key(p, r)] = c * apr - s * aqr
    A[key(q, r)] = s * apr + c * aqr
    for i in range(3):
        vip, viq = V[(i, p)], V[(i, q)]
        V[(i, p)] = c * vip - s * viq
        V[(i, q)] = s * vip + c * viq


def _normals_from_moments(m_ref):
    inv_k = 1.0 / K
    sx, sy, sz = m_ref[0], m_ref[1], m_ref[2]
    sxx, syy, szz = m_ref[3], m_ref[4], m_ref[5]
    sxy, sxz, syz = m_ref[6], m_ref[7], m_ref[8]
    mx, my, mz = sx * inv_k, sy * inv_k, sz * inv_k
    A = {
        (0, 0): sxx * inv_k - mx * mx,
        (1, 1): syy * inv_k - my * my,
        (2, 2): szz * inv_k - mz * mz,
        (0, 1): sxy * inv_k - mx * my,
        (0, 2): sxz * inv_k - mx * mz,
        (1, 2): syz * inv_k - my * mz,
    }
    one = jnp.ones_like(sx)
    zero = jnp.zeros_like(sx)
    V = {(i, j): (one if i == j else zero)
         for i in range(3) for j in range(3)}
    for _ in range(NSWEEP):
        for (p, q) in ((0, 2), (1, 2), (0, 1)):
            _rotate(A, V, p, q)
    d0, d1, d2 = A[(0, 0)], A[(1, 1)], A[(2, 2)]
    take0 = (d0 <= d1) & (d0 <= d2)
    take1 = jnp.logical_not(take0) & (d1 <= d2)

    def pick(i):
        return jnp.where(take0, V[(i, 0)],
                         jnp.where(take1, V[(i, 1)], V[(i, 2)]))

    nx, ny, nz = pick(0), pick(1), pick(2)
    nrm = jnp.sqrt(nx * nx + ny * ny + nz * nz) + 1e-12
    return nx / nrm, ny / nrm, nz / nrm


def _loss_kernel(mp_ref, mg_ref, out_ref):
    px, py, pz = _normals_from_moments(mp_ref)
    gx, gy, gz = _normals_from_moments(mg_ref)
    dot = px * gx + py * gy + pz * gz
    out_ref[0, 0] = 1.0 - jnp.sum(dot) / (B * N)


def _loss(mom_p, mom_g):
    out = pl.pallas_call(
        _loss_kernel,
        out_specs=pl.BlockSpec(memory_space=pltpu.SMEM),
        out_shape=jax.ShapeDtypeStruct((1, 1), jnp.float32),
    )(mom_p, mom_g)
    return out.reshape(())


@jax.jit
def kernel(pred, gt):
    pts = jnp.concatenate([pred, gt], axis=0)
    pts8 = jnp.pad(pts, ((0, 0), (0, 0), (0, 5)))
    idx, mtab = _knn_idx(pts8)
    idx3 = idx.reshape(NW, 128, 128)
    mom = _sc_moments(mtab.reshape(M_ALL, 16), idx3)
    mom2 = mom.reshape(2, B * N, 16).transpose(0, 2, 1)
    mom2 = mom2.reshape(2, 16, 128, 128)
    return _loss(mom2[0], mom2[1])
